# Initial kernel scaffold; baseline (speedup 1.0000x reference)
import jax
import jax.numpy as jnp
from jax.experimental import pallas as pl


def kernel(batch, alphas, p0, ctx1_keys, ctx1_probs, ctx2_keys, ctx2_probs):
    raise NotImplementedError("write your pallas kernel here")


# speedup vs baseline: 1.3121x; 1.3121x over previous
import functools
import math

import jax
import jax.numpy as jnp
from jax import lax
from jax.experimental import pallas as pl
from jax.experimental.pallas import tpu as pltpu
from jax.experimental.pallas import tpu_sc as plsc

_NC = 2
_NS = 16
_NW = _NC * _NS
_L = 16

_INT_MAX = jnp.iinfo(jnp.int32).max


def _bisect(keys_ref, q, num_keys, steps):
    lo = jnp.zeros((_L,), jnp.int32)
    hi = jnp.full((_L,), num_keys, jnp.int32)
    for _ in range(steps):
        mid = lax.shift_right_arithmetic(lo + hi, 1)
        kv = plsc.load_gather(keys_ref, [mid])
        pred = kv < q
        lo = jnp.where(pred, mid + 1, lo)
        hi = jnp.where(pred, hi, mid)
    idxc = jnp.minimum(lo, num_keys - 1)
    kv = plsc.load_gather(keys_ref, [idxc])
    return idxc, kv == q


def _make_sc_lookup(n, seq_len, vocab, c1, c2):
    chunk = n // _NW
    steps1 = max(1, math.ceil(math.log2(c1 + 1)))
    steps2 = max(1, math.ceil(math.log2(c2 + 1)))
    pad1 = c1 + _L
    pad2 = c2 + _L
    mesh = plsc.VectorSubcoreMesh(
        core_axis_name="c", subcore_axis_name="s",
        num_cores=_NC, num_subcores=_NS)
    out_sds = jax.ShapeDtypeStruct((n,), jnp.int32)

    @functools.partial(
        pl.kernel,
        out_type=(out_sds, out_sds, out_sds, out_sds),
        mesh=mesh,
        compiler_params=pltpu.CompilerParams(needs_layout_passes=False),
        scratch_types=[
            pltpu.VMEM((chunk + 8,), jnp.int32),
            pltpu.VMEM((pad1,), jnp.int32),
            pltpu.VMEM((pad2,), jnp.int32),
            pltpu.VMEM((chunk,), jnp.int32),
            pltpu.VMEM((chunk,), jnp.int32),
            pltpu.VMEM((chunk,), jnp.int32),
            pltpu.VMEM((chunk,), jnp.int32),
        ],
    )
    def sc_lookup(batch_hbm, k1_hbm, k2_hbm,
                  i1_hbm, f1_hbm, i2_hbm, f2_hbm,
                  qbuf, k1v, k2v, o_i1, o_f1, o_i2, o_f2):
        wid = lax.axis_index("s") * _NC + lax.axis_index("c")
        base = wid * chunk
        pltpu.sync_copy(k1_hbm, k1v.at[pl.ds(0, c1)])
        pltpu.sync_copy(k2_hbm, k2v.at[pl.ds(0, c2)])
        k1v[pl.ds(c1, _L)] = jnp.full((_L,), _INT_MAX, jnp.int32)
        k2v[pl.ds(c2, _L)] = jnp.full((_L,), _INT_MAX, jnp.int32)
        pltpu.sync_copy(batch_hbm.at[pl.ds(base, chunk)],
                        qbuf.at[pl.ds(8, chunk)])
        @pl.when(wid > 0)
        def _():
            pltpu.sync_copy(batch_hbm.at[pl.ds(base - 8, 8)],
                            qbuf.at[pl.ds(0, 8)])

        iota = lax.iota(jnp.int32, _L)

        def step(t, carry):
            off = t * _L
            j = lax.rem(base + off, seq_len) + iota
            idxs = off + 8 + iota
            q1 = plsc.load_gather(qbuf, [idxs - 1])
            t2 = plsc.load_gather(qbuf, [idxs - 2])
            valid1 = j >= 1
            valid2 = j >= 2
            i1, hit1 = _bisect(k1v, q1, c1, steps1)
            q2 = t2 * vocab + q1
            i2, hit2 = _bisect(k2v, q2, c2, steps2)
            f1 = (hit1 & valid1).astype(jnp.int32)
            f2 = (hit2 & valid2).astype(jnp.int32)
            o_i1[pl.ds(off, _L)] = jnp.where(f1 > 0, i1, 0)
            o_f1[pl.ds(off, _L)] = f1
            o_i2[pl.ds(off, _L)] = jnp.where(f2 > 0, i2, 0)
            o_f2[pl.ds(off, _L)] = f2
            return carry

        lax.fori_loop(0, chunk // _L, step, 0)
        pltpu.sync_copy(o_i1, i1_hbm.at[pl.ds(base, chunk)])
        pltpu.sync_copy(o_f1, f1_hbm.at[pl.ds(base, chunk)])
        pltpu.sync_copy(o_i2, i2_hbm.at[pl.ds(base, chunk)])
        pltpu.sync_copy(o_f2, f2_hbm.at[pl.ds(base, chunk)])

    return sc_lookup


_P = 8


def _tc_body(i1s, f1s, i2s, f2s, p0_ref, al_ref, *refs):
    rows1 = refs[:_P]
    rows2 = refs[_P:2 * _P]
    out_ref = refs[2 * _P]
    g = pl.program_id(0)
    a0 = al_ref[0]
    a1 = al_ref[1]
    a2 = al_ref[2]
    vocab = out_ref.shape[1]
    base = a0 * p0_ref[0, :]
    for k in range(_P):
        p = g * _P + k
        c1 = a1 * f1s[p].astype(jnp.float32)
        c2 = a2 * f2s[p].astype(jnp.float32)
        miss = (a1 - c1 + a2 - c2) * (1.0 / vocab)
        row = (base + miss) + c1 * rows1[k][0, 0, :] + c2 * rows2[k][0, 0, :]
        out_ref[k, :] = jnp.log(row)


def _make_tc_combine(n, vocab, c1, c2):
    def im1(k, g, i1, f1, i2, f2):
        return (i1[g * _P + k], 0, 0)

    def im2(k, g, i1, f1, i2, f2):
        return (i2[g * _P + k], 0, 0)

    grid_spec = pltpu.PrefetchScalarGridSpec(
        num_scalar_prefetch=4,
        grid=(n // _P,),
        in_specs=[
            pl.BlockSpec((1, vocab), lambda g, *s: (0, 0)),
            pl.BlockSpec(memory_space=pltpu.SMEM),
            *[pl.BlockSpec((1, 1, vocab), functools.partial(im1, k))
              for k in range(_P)],
            *[pl.BlockSpec((1, 1, vocab), functools.partial(im2, k))
              for k in range(_P)],
        ],
        out_specs=pl.BlockSpec((_P, vocab), lambda g, *s: (g, 0)),
    )
    return pl.pallas_call(
        _tc_body,
        grid_spec=grid_spec,
        out_shape=jax.ShapeDtypeStruct((n, vocab), jnp.float32),
        compiler_params=pltpu.CompilerParams(
            dimension_semantics=("arbitrary",)),
    )


def kernel(batch, alphas, p0, ctx1_keys, ctx1_probs, ctx2_keys, ctx2_probs):
    b, s = batch.shape
    vocab = p0.shape[0]
    c1 = ctx1_keys.shape[0]
    c2 = ctx2_keys.shape[0]
    n = b * s
    flat = batch.reshape(n).astype(jnp.int32)

    sc_lookup = _make_sc_lookup(n, s, vocab, c1, c2)
    i1, f1, i2, f2 = sc_lookup(flat, ctx1_keys, ctx2_keys)

    tc_combine = _make_tc_combine(n, vocab, c1, c2)
    p0_2d = p0.reshape(1, vocab)
    r1 = ctx1_probs.reshape(c1, 1, vocab)
    r2 = ctx2_probs.reshape(c2, 1, vocab)
    out = tc_combine(i1, f1, i2, f2, p0_2d, alphas,
                     *([r1] * _P), *([r2] * _P))
    return out.reshape(b, s, vocab)


# speedup vs baseline: 1.8139x; 1.3824x over previous
import functools
import math

import jax
import jax.numpy as jnp
from jax import lax
from jax.experimental import pallas as pl
from jax.experimental.pallas import tpu as pltpu
from jax.experimental.pallas import tpu_sc as plsc

_NC = 2
_NS = 16
_NW = _NC * _NS
_L = 16

_INT_MAX = jnp.iinfo(jnp.int32).max


def _bisect(keys_ref, q, num_keys, steps):
    lo = jnp.zeros((_L,), jnp.int32)
    hi = jnp.full((_L,), num_keys, jnp.int32)
    for _ in range(steps):
        mid = lax.shift_right_arithmetic(lo + hi, 1)
        kv = plsc.load_gather(keys_ref, [mid])
        pred = kv < q
        lo = jnp.where(pred, mid + 1, lo)
        hi = jnp.where(pred, hi, mid)
    idxc = jnp.minimum(lo, num_keys - 1)
    kv = plsc.load_gather(keys_ref, [idxc])
    return idxc, kv == q


def _make_sc_lookup(n, seq_len, vocab, c1, c2):
    chunk = n // _NW
    steps1 = max(1, math.ceil(math.log2(c1 + 1)))
    steps2 = max(1, math.ceil(math.log2(c2 + 1)))
    pad1 = c1 + _L
    pad2 = c2 + _L
    mesh = plsc.VectorSubcoreMesh(
        core_axis_name="c", subcore_axis_name="s",
        num_cores=_NC, num_subcores=_NS)
    out_sds = jax.ShapeDtypeStruct((n,), jnp.int32)

    @functools.partial(
        pl.kernel,
        out_type=(out_sds, out_sds, out_sds, out_sds),
        mesh=mesh,
        compiler_params=pltpu.CompilerParams(needs_layout_passes=False),
        scratch_types=[
            pltpu.VMEM((chunk + 8,), jnp.int32),
            pltpu.VMEM((pad1,), jnp.int32),
            pltpu.VMEM((pad2,), jnp.int32),
            pltpu.VMEM((chunk,), jnp.int32),
            pltpu.VMEM((chunk,), jnp.int32),
            pltpu.VMEM((chunk,), jnp.int32),
            pltpu.VMEM((chunk,), jnp.int32),
        ],
    )
    def sc_lookup(batch_hbm, k1_hbm, k2_hbm,
                  i1_hbm, f1_hbm, i2_hbm, f2_hbm,
                  qbuf, k1v, k2v, o_i1, o_f1, o_i2, o_f2):
        wid = lax.axis_index("s") * _NC + lax.axis_index("c")
        base = wid * chunk
        pltpu.sync_copy(k1_hbm, k1v.at[pl.ds(0, c1)])
        pltpu.sync_copy(k2_hbm, k2v.at[pl.ds(0, c2)])
        k1v[pl.ds(c1, _L)] = jnp.full((_L,), _INT_MAX, jnp.int32)
        k2v[pl.ds(c2, _L)] = jnp.full((_L,), _INT_MAX, jnp.int32)
        pltpu.sync_copy(batch_hbm.at[pl.ds(base, chunk)],
                        qbuf.at[pl.ds(8, chunk)])
        @pl.when(wid > 0)
        def _():
            pltpu.sync_copy(batch_hbm.at[pl.ds(base - 8, 8)],
                            qbuf.at[pl.ds(0, 8)])

        iota = lax.iota(jnp.int32, _L)

        def step(t, carry):
            off = t * _L
            j = lax.rem(base + off, seq_len) + iota
            idxs = off + 8 + iota
            q1 = plsc.load_gather(qbuf, [idxs - 1])
            t2 = plsc.load_gather(qbuf, [idxs - 2])
            valid1 = j >= 1
            valid2 = j >= 2
            i1, hit1 = _bisect(k1v, q1, c1, steps1)
            q2 = t2 * vocab + q1
            i2, hit2 = _bisect(k2v, q2, c2, steps2)
            f1 = (hit1 & valid1).astype(jnp.int32)
            f2 = (hit2 & valid2).astype(jnp.int32)
            o_i1[pl.ds(off, _L)] = jnp.where(f1 > 0, i1, 0)
            o_f1[pl.ds(off, _L)] = f1
            o_i2[pl.ds(off, _L)] = jnp.where(f2 > 0, i2, 0)
            o_f2[pl.ds(off, _L)] = f2
            return carry

        lax.fori_loop(0, chunk // _L, step, 0)
        pltpu.sync_copy(o_i1, i1_hbm.at[pl.ds(base, chunk)])
        pltpu.sync_copy(o_f1, f1_hbm.at[pl.ds(base, chunk)])
        pltpu.sync_copy(o_i2, i2_hbm.at[pl.ds(base, chunk)])
        pltpu.sync_copy(o_f2, f2_hbm.at[pl.ds(base, chunk)])

    return sc_lookup


_P = 16


def _tc_body(i1s, f1s, i2s, f2s, p0_ref, al_ref, *refs):
    rows1 = refs[:_P]
    rows2 = refs[_P:2 * _P]
    out_ref = refs[2 * _P]
    g = pl.program_id(0)
    a0 = al_ref[0]
    a1 = al_ref[1]
    a2 = al_ref[2]
    vocab = out_ref.shape[1]
    base = a0 * p0_ref[0, :]
    for k in range(_P):
        p = g * _P + k
        c1 = a1 * f1s[p].astype(jnp.float32)
        c2 = a2 * f2s[p].astype(jnp.float32)
        miss = (a1 - c1 + a2 - c2) * (1.0 / vocab)
        row = (base + miss) + c1 * rows1[k][0, 0, :] + c2 * rows2[k][0, 0, :]
        out_ref[k, :] = jnp.log(row)


def _make_tc_combine(n, vocab, c1, c2):
    def im1(k, g, i1, f1, i2, f2):
        return (i1[g * _P + k], 0, 0)

    def im2(k, g, i1, f1, i2, f2):
        return (i2[g * _P + k], 0, 0)

    grid_spec = pltpu.PrefetchScalarGridSpec(
        num_scalar_prefetch=4,
        grid=(n // _P,),
        in_specs=[
            pl.BlockSpec((1, vocab), lambda g, *s: (0, 0)),
            pl.BlockSpec(memory_space=pltpu.SMEM),
            *[pl.BlockSpec((1, 1, vocab), functools.partial(im1, k))
              for k in range(_P)],
            *[pl.BlockSpec((1, 1, vocab), functools.partial(im2, k))
              for k in range(_P)],
        ],
        out_specs=pl.BlockSpec((_P, vocab), lambda g, *s: (g, 0)),
    )
    return pl.pallas_call(
        _tc_body,
        grid_spec=grid_spec,
        out_shape=jax.ShapeDtypeStruct((n, vocab), jnp.float32),
        compiler_params=pltpu.CompilerParams(
            dimension_semantics=("arbitrary",)),
    )


def kernel(batch, alphas, p0, ctx1_keys, ctx1_probs, ctx2_keys, ctx2_probs):
    b, s = batch.shape
    vocab = p0.shape[0]
    c1 = ctx1_keys.shape[0]
    c2 = ctx2_keys.shape[0]
    n = b * s
    flat = batch.reshape(n).astype(jnp.int32)

    sc_lookup = _make_sc_lookup(n, s, vocab, c1, c2)
    i1, f1, i2, f2 = sc_lookup(flat, ctx1_keys, ctx2_keys)

    tc_combine = _make_tc_combine(n, vocab, c1, c2)
    p0_2d = p0.reshape(1, vocab)
    r1 = ctx1_probs.reshape(c1, 1, vocab)
    r2 = ctx2_probs.reshape(c2, 1, vocab)
    out = tc_combine(i1, f1, i2, f2, p0_2d, alphas,
                     *([r1] * _P), *([r2] * _P))
    return out.reshape(b, s, vocab)
